# baseline (speedup 1.0000x reference)
import dataclasses
import functools

import jax
import jax.numpy as jnp
from jax import lax
from jax.experimental import pallas as pl
from jax.experimental.pallas import tpu as pltpu
from jax.experimental.pallas import tpu_sc as plsc

B = 1024
L = 200
EMB = 300
NUM_LABELS = 32768
VOCAB = 100001

NC = 2
NS = 16
NW = NC * NS
ROWS_PER_W = B // NW
NFULL = EMB // 16
LH0 = 104
LH1 = L - LH0

_vector_mesh = plsc.VectorSubcoreMesh(core_axis_name="c", subcore_axis_name="s")

_sc_params = pltpu.CompilerParams()
if "needs_layout_passes" in pltpu.CompilerParams.__dataclass_fields__:
    _sc_params = dataclasses.replace(_sc_params, needs_layout_passes=False)
if "use_tc_tiling_on_sc" in pltpu.CompilerParams.__dataclass_fields__:
    _sc_params = dataclasses.replace(_sc_params, use_tc_tiling_on_sc=False)


RB = 1024


S2_OFF = EMB - 128


def _repack_body(t_ref, o0_ref, o1_ref, o2_ref):
    v = jnp.transpose(t_ref[...]).astype(jnp.bfloat16)
    o0_ref[...] = v[:, 0:128]
    o1_ref[...] = v[:, 128:256]
    o2_ref[...] = v[:, S2_OFF:S2_OFF + 128]


def _repack(emb_table_t):
    grid = ((VOCAB + RB - 1) // RB,)
    out = jax.ShapeDtypeStruct((VOCAB, 128), jnp.bfloat16)
    return pl.pallas_call(
        _repack_body,
        grid=grid,
        in_specs=[pl.BlockSpec((EMB + 4, RB), lambda i: (0, i))],
        out_specs=[pl.BlockSpec((RB, 128), lambda i: (i, 0))] * 3,
        out_shape=[out, out, out],
    )(emb_table_t)


def _splat16(ref, i, j):
    return plsc.load_gather(
        ref, [jnp.full((16,), i, jnp.int32), jnp.full((16,), j, jnp.int32)]
    )


@functools.partial(
    pl.kernel,
    out_type=jax.ShapeDtypeStruct((B, EMB), jnp.float32),
    mesh=_vector_mesh,
    scratch_types=[
        pltpu.VMEM((ROWS_PER_W, L), jnp.int32),
        pltpu.VMEM((ROWS_PER_W, L), jnp.float32),
        pltpu.VMEM((LH0, 128), jnp.bfloat16),
        pltpu.VMEM((LH0, 128), jnp.bfloat16),
        pltpu.VMEM((LH0, 128), jnp.bfloat16),
        pltpu.VMEM((LH1, 128), jnp.bfloat16),
        pltpu.VMEM((LH1, 128), jnp.bfloat16),
        pltpu.VMEM((LH1, 128), jnp.bfloat16),
        pltpu.VMEM((ROWS_PER_W, EMB), jnp.float32),
        pltpu.SemaphoreType.DMA,
        pltpu.SemaphoreType.DMA,
    ],
    compiler_params=_sc_params,
)
def _sc_embed(x_hbm, w_hbm, t0_hbm, t1_hbm, t2_hbm, rep_hbm,
              xv, wv, a0, a1, a2, b0, b1, b2, repst, semA, semB):
    wid = lax.axis_index("s") * NC + lax.axis_index("c")
    base = wid * ROWS_PER_W
    pltpu.sync_copy(x_hbm.at[pl.ds(base, ROWS_PER_W)], xv)
    pltpu.sync_copy(w_hbm.at[pl.ds(base, ROWS_PER_W)], wv)
    iota = lax.iota(jnp.int32, 16)

    def start(b, off, n, g0, g1, g2, sem):
        idx = xv.at[b, pl.ds(off, n)]
        pltpu.make_async_copy(t0_hbm.at[idx], g0, sem).start()
        pltpu.make_async_copy(t1_hbm.at[idx], g1, sem).start()
        pltpu.make_async_copy(t2_hbm.at[idx], g2, sem).start()

    def wait(b, off, n, g0, g1, g2, sem):
        idx = xv.at[b, pl.ds(off, n)]
        pltpu.make_async_copy(t0_hbm.at[idx], g0, sem).wait()
        pltpu.make_async_copy(t1_hbm.at[idx], g1, sem).wait()
        pltpu.make_async_copy(t2_hbm.at[idx], g2, sem).wait()

    G_SRC = (
        [(0, 32 * g) for g in range(4)]
        + [(1, 32 * g) for g in range(4)]
        + [(2, 256 - S2_OFF), (2, (EMB - 32) - S2_OFF)]
    )
    NG = len(G_SRC)

    def accumulate(g0, g1, g2, b, off, n, accs):
        bufs = (g0, g1, g2)

        def lbody(ll, accs):
            l = off + ll
            w_vec = _splat16(wv, b, l)
            new = list(accs)
            for g, (slab, goff) in enumerate(G_SRC):
                p = bufs[slab][ll, pl.ds(goff, 32)]
                e, o = plsc.unpack(
                    p, format=plsc.PackFormat.INTERLEAVED,
                    preferred_element_type=jnp.float32,
                )
                new[2 * g] = accs[2 * g] + w_vec * e
                new[2 * g + 1] = accs[2 * g + 1] + w_vec * o
            return tuple(new)

        return lax.fori_loop(0, n, lbody, accs)

    zeros = tuple(jnp.zeros((16,), jnp.float32) for _ in range(20))

    start(0, 0, LH0, a0, a1, a2, semA)

    @pl.loop(0, ROWS_PER_W)
    def _row(b):
        start(b, LH0, LH1, b0, b1, b2, semB)
        wait(b, 0, LH0, a0, a1, a2, semA)
        accs = accumulate(a0, a1, a2, b, 0, LH0, zeros)

        @pl.when(b < ROWS_PER_W - 1)
        def _():
            start(b + 1, 0, LH0, a0, a1, a2, semA)

        wait(b, LH0, LH1, b0, b1, b2, semB)
        accs = accumulate(b0, b1, b2, b, LH0, LH1, accs)

        brow = jnp.full((16,), b, jnp.int32)
        for g, base in enumerate((0, 32, 64, 96, 128, 160, 192, 224, 256, 268)):
            cols = base + 2 * iota
            plsc.store_scatter(repst, [brow, cols], accs[2 * g])
            plsc.store_scatter(repst, [brow, cols + 1], accs[2 * g + 1])

    pltpu.sync_copy(repst, rep_hbm.at[pl.ds(base, ROWS_PER_W)])


BN = 2048


def _tc_body(rep_ref, w_ref, b_ref, out_ref):
    r = jnp.maximum(rep_ref[...], 0.0).astype(jnp.bfloat16)
    w = w_ref[...].astype(jnp.bfloat16)
    acc = lax.dot_general(
        r, w, (((1,), (0,)), ((), ())), preferred_element_type=jnp.float32
    )
    out_ref[...] = acc + b_ref[...]


def _tc_classify(rep, clf_W, clf_b):
    return pl.pallas_call(
        _tc_body,
        grid=(NUM_LABELS // BN,),
        in_specs=[
            pl.BlockSpec((B, EMB), lambda i: (0, 0)),
            pl.BlockSpec((EMB, BN), lambda i: (0, i)),
            pl.BlockSpec((1, BN), lambda i: (0, i)),
        ],
        out_specs=pl.BlockSpec((B, BN), lambda i: (0, i)),
        out_shape=jax.ShapeDtypeStruct((B, NUM_LABELS), jnp.float32),
    )(rep, clf_W.T, clf_b.reshape(1, NUM_LABELS))


def kernel(X, X_w, emb_table, clf_W, clf_b):
    X = X.astype(jnp.int32)
    t0, t1, t2 = _repack(emb_table.T)
    rep = _sc_embed(X, X_w, t0, t1, t2)
    return _tc_classify(rep, clf_W, clf_b)


# speedup vs baseline: 1.9546x; 1.9546x over previous
import dataclasses
import functools

import jax
import jax.numpy as jnp
from jax import lax
from jax.experimental import pallas as pl
from jax.experimental.pallas import tpu as pltpu
from jax.experimental.pallas import tpu_sc as plsc

B = 1024
L = 200
EMB = 300
NUM_LABELS = 32768
VOCAB = 100001

NC = 2
NS = 16
NW = NC * NS
ROWS_PER_W = B // NW
LH0 = 104
LH1 = L - LH0

HALF = 152
S1_OFF = 24

_vector_mesh = plsc.VectorSubcoreMesh(core_axis_name="c", subcore_axis_name="s")

_sc_params = pltpu.CompilerParams()
if "needs_layout_passes" in pltpu.CompilerParams.__dataclass_fields__:
    _sc_params = dataclasses.replace(_sc_params, needs_layout_passes=False)
if "use_tc_tiling_on_sc" in pltpu.CompilerParams.__dataclass_fields__:
    _sc_params = dataclasses.replace(_sc_params, use_tc_tiling_on_sc=False)


RB = 1024


def _repack_body(t_ref, o0_ref, o1_ref):
    v = jnp.transpose(t_ref[...])
    lo = lax.bitcast_convert_type(
        v[:, 0:HALF].astype(jnp.bfloat16), jnp.uint16
    ).astype(jnp.uint32)
    hi = lax.bitcast_convert_type(
        v[:, HALF:2 * HALF].astype(jnp.bfloat16), jnp.uint16
    ).astype(jnp.uint32)
    packed = lax.bitwise_or(lo, lax.shift_left(hi, jnp.uint32(16)))
    o0_ref[...] = packed[:, 0:128]
    o1_ref[...] = packed[:, S1_OFF:S1_OFF + 128]


def _repack(emb_table_t):
    grid = ((VOCAB + RB - 1) // RB,)
    out = jax.ShapeDtypeStruct((VOCAB, 128), jnp.uint32)
    return pl.pallas_call(
        _repack_body,
        grid=grid,
        in_specs=[pl.BlockSpec((2 * HALF, RB), lambda i: (0, i))],
        out_specs=[pl.BlockSpec((RB, 128), lambda i: (i, 0))] * 2,
        out_shape=[out, out],
    )(emb_table_t)


def _splat16(ref, i, j):
    return plsc.load_gather(
        ref, [jnp.full((16,), i, jnp.int32), jnp.full((16,), j, jnp.int32)]
    )


def _unpack32(p):
    b16 = plsc.bitcast(p, jnp.bfloat16)
    return plsc.unpack(
        b16, format=plsc.PackFormat.INTERLEAVED,
        preferred_element_type=jnp.float32,
    )


@functools.partial(
    pl.kernel,
    out_type=jax.ShapeDtypeStruct((B, EMB), jnp.float32),
    mesh=_vector_mesh,
    scratch_types=[
        pltpu.VMEM((ROWS_PER_W, L), jnp.int32),
        pltpu.VMEM((ROWS_PER_W, L), jnp.float32),
        pltpu.VMEM((LH0, 128), jnp.uint32),
        pltpu.VMEM((LH0, 128), jnp.uint32),
        pltpu.VMEM((LH1, 128), jnp.uint32),
        pltpu.VMEM((LH1, 128), jnp.uint32),
        pltpu.VMEM((ROWS_PER_W, EMB), jnp.float32),
        pltpu.SemaphoreType.DMA,
        pltpu.SemaphoreType.DMA,
    ],
    compiler_params=_sc_params,
)
def _sc_embed(x_hbm, w_hbm, t0_hbm, t1_hbm, rep_hbm,
              xv, wv, a0, a1, b0, b1, repst, semA, semB):
    wid = lax.axis_index("s") * NC + lax.axis_index("c")
    base = wid * ROWS_PER_W
    pltpu.sync_copy(x_hbm.at[pl.ds(base, ROWS_PER_W)], xv)
    pltpu.sync_copy(w_hbm.at[pl.ds(base, ROWS_PER_W)], wv)

    def start(b, off, n, g0, g1, sem):
        idx = xv.at[b, pl.ds(off, n)]
        pltpu.make_async_copy(t0_hbm.at[idx], g0, sem).start()
        pltpu.make_async_copy(t1_hbm.at[idx], g1, sem).start()

    def wait(b, off, n, g0, g1, sem):
        idx = xv.at[b, pl.ds(off, n)]
        pltpu.make_async_copy(t0_hbm.at[idx], g0, sem).wait()
        pltpu.make_async_copy(t1_hbm.at[idx], g1, sem).wait()

    def accumulate(g0, g1, b, off, n, accs):
        def lbody(ll, accs):
            l = off + ll
            w_vec = _splat16(wv, b, l)
            new = list(accs)
            for j in range(9):
                if j < 8:
                    p = g0[ll, pl.ds(16 * j, 16)]
                else:
                    p = g1[ll, pl.ds(128 - S1_OFF, 16)]
                e, o = _unpack32(p)
                new[j] = accs[j] + w_vec * e
                new[9 + j] = accs[9 + j] + w_vec * o
            et, _ = _unpack32(g1[ll, pl.ds(136 - S1_OFF, 16)])
            new[18] = accs[18] + w_vec * et
            _, ot = _unpack32(g1[ll, pl.ds(132 - S1_OFF, 16)])
            new[19] = accs[19] + w_vec * ot
            return tuple(new)

        return lax.fori_loop(0, n, lbody, accs)

    zeros = tuple(jnp.zeros((16,), jnp.float32) for _ in range(20))

    start(0, 0, LH0, a0, a1, semA)

    @pl.loop(0, ROWS_PER_W)
    def _row(b):
        start(b, LH0, LH1, b0, b1, semB)
        wait(b, 0, LH0, a0, a1, semA)
        accs = accumulate(a0, a1, b, 0, LH0, zeros)

        @pl.when(b < ROWS_PER_W - 1)
        def _():
            start(b + 1, 0, LH0, a0, a1, semA)

        wait(b, LH0, LH1, b0, b1, semB)
        accs = accumulate(b0, b1, b, LH0, LH1, accs)

        for j in range(9):
            repst[b, pl.ds(16 * j, 16)] = accs[j]
        for j in range(9):
            repst[b, pl.ds(HALF + 16 * j, 16)] = accs[9 + j]
        repst[b, pl.ds(136, 16)] = accs[18]
        repst[b, pl.ds(EMB - 16, 16)] = accs[19]

    pltpu.sync_copy(repst, rep_hbm.at[pl.ds(base, ROWS_PER_W)])


BN = 2048


def _tc_body(rep_ref, w_ref, b_ref, out_ref):
    r = jnp.maximum(rep_ref[...], 0.0).astype(jnp.bfloat16)
    w = w_ref[...].astype(jnp.bfloat16)
    acc = lax.dot_general(
        r, w, (((1,), (0,)), ((), ())), preferred_element_type=jnp.float32
    )
    out_ref[...] = acc + b_ref[...]


def _tc_classify(rep, clf_W, clf_b):
    return pl.pallas_call(
        _tc_body,
        grid=(NUM_LABELS // BN,),
        in_specs=[
            pl.BlockSpec((B, EMB), lambda i: (0, 0)),
            pl.BlockSpec((EMB, BN), lambda i: (0, i)),
            pl.BlockSpec((1, BN), lambda i: (0, i)),
        ],
        out_specs=pl.BlockSpec((B, BN), lambda i: (0, i)),
        out_shape=jax.ShapeDtypeStruct((B, NUM_LABELS), jnp.float32),
    )(rep, clf_W.T, clf_b.reshape(1, NUM_LABELS))


def kernel(X, X_w, emb_table, clf_W, clf_b):
    X = X.astype(jnp.int32)
    t0, t1 = _repack(emb_table.T)
    rep = _sc_embed(X, X_w, t0, t1)
    return _tc_classify(rep, clf_W, clf_b)


# speedup vs baseline: 2.1588x; 1.1045x over previous
import dataclasses
import functools

import jax
import jax.numpy as jnp
from jax import lax
from jax.experimental import pallas as pl
from jax.experimental.pallas import tpu as pltpu
from jax.experimental.pallas import tpu_sc as plsc

B = 1024
L = 200
EMB = 300
NUM_LABELS = 32768
VOCAB = 100001

NC = 2
NS = 16
NW = NC * NS
ROWS_PER_W = B // NW
LH0 = 104
LH1 = L - LH0

HALF = 152
S1_OFF = 24

_vector_mesh = plsc.VectorSubcoreMesh(core_axis_name="c", subcore_axis_name="s")

_sc_params = pltpu.CompilerParams()
if "needs_layout_passes" in pltpu.CompilerParams.__dataclass_fields__:
    _sc_params = dataclasses.replace(_sc_params, needs_layout_passes=False)
if "use_tc_tiling_on_sc" in pltpu.CompilerParams.__dataclass_fields__:
    _sc_params = dataclasses.replace(_sc_params, use_tc_tiling_on_sc=False)


RB = 2048


def _repack_body(t_ref, o0_ref, o1_ref):
    v = jnp.transpose(t_ref[...])
    lo = lax.bitcast_convert_type(
        v[:, 0:HALF].astype(jnp.bfloat16), jnp.uint16
    ).astype(jnp.uint32)
    hi = lax.bitcast_convert_type(
        v[:, HALF:2 * HALF].astype(jnp.bfloat16), jnp.uint16
    ).astype(jnp.uint32)
    packed = lax.bitwise_or(lo, lax.shift_left(hi, jnp.uint32(16)))
    o0_ref[...] = packed[:, 0:128]
    o1_ref[...] = packed[:, S1_OFF:S1_OFF + 128]


def _repack(emb_table_t):
    grid = ((VOCAB + RB - 1) // RB,)
    out = jax.ShapeDtypeStruct((VOCAB, 128), jnp.uint32)
    return pl.pallas_call(
        _repack_body,
        grid=grid,
        in_specs=[pl.BlockSpec((2 * HALF, RB), lambda i: (0, i))],
        out_specs=[pl.BlockSpec((RB, 128), lambda i: (i, 0))] * 2,
        out_shape=[out, out],
    )(emb_table_t)


def _splat16(ref, i, j):
    return plsc.load_gather(
        ref, [jnp.full((16,), i, jnp.int32), jnp.full((16,), j, jnp.int32)]
    )


def _unpack32(p):
    b16 = plsc.bitcast(p, jnp.bfloat16)
    return plsc.unpack(
        b16, format=plsc.PackFormat.INTERLEAVED,
        preferred_element_type=jnp.float32,
    )


@functools.partial(
    pl.kernel,
    out_type=jax.ShapeDtypeStruct((B, EMB), jnp.float32),
    mesh=_vector_mesh,
    scratch_types=[
        pltpu.VMEM((ROWS_PER_W, L), jnp.int32),
        pltpu.VMEM((ROWS_PER_W, L), jnp.float32),
        pltpu.VMEM((LH0, 128), jnp.uint32),
        pltpu.VMEM((LH0, 128), jnp.uint32),
        pltpu.VMEM((LH1, 128), jnp.uint32),
        pltpu.VMEM((LH1, 128), jnp.uint32),
        pltpu.VMEM((ROWS_PER_W, EMB), jnp.float32),
        pltpu.SemaphoreType.DMA,
        pltpu.SemaphoreType.DMA,
    ],
    compiler_params=_sc_params,
)
def _sc_embed(x_hbm, w_hbm, t0_hbm, t1_hbm, rep_hbm,
              xv, wv, a0, a1, b0, b1, repst, semA, semB):
    wid = lax.axis_index("s") * NC + lax.axis_index("c")
    base = wid * ROWS_PER_W
    pltpu.sync_copy(x_hbm.at[pl.ds(base, ROWS_PER_W)], xv)
    pltpu.sync_copy(w_hbm.at[pl.ds(base, ROWS_PER_W)], wv)

    def start(b, off, n, g0, g1, sem):
        idx = xv.at[b, pl.ds(off, n)]
        pltpu.make_async_copy(t0_hbm.at[idx], g0, sem).start()
        pltpu.make_async_copy(t1_hbm.at[idx], g1, sem).start()

    def wait(b, off, n, g0, g1, sem):
        idx = xv.at[b, pl.ds(off, n)]
        pltpu.make_async_copy(t0_hbm.at[idx], g0, sem).wait()
        pltpu.make_async_copy(t1_hbm.at[idx], g1, sem).wait()

    def accumulate(g0, g1, b, off, n, accs):
        def lbody(ll, accs):
            l = off + ll
            w_vec = _splat16(wv, b, l)
            new = list(accs)
            for j in range(9):
                if j < 8:
                    p = g0[ll, pl.ds(16 * j, 16)]
                else:
                    p = g1[ll, pl.ds(128 - S1_OFF, 16)]
                e, o = _unpack32(p)
                new[j] = accs[j] + w_vec * e
                new[9 + j] = accs[9 + j] + w_vec * o
            et, _ = _unpack32(g1[ll, pl.ds(136 - S1_OFF, 16)])
            new[18] = accs[18] + w_vec * et
            _, ot = _unpack32(g1[ll, pl.ds(132 - S1_OFF, 16)])
            new[19] = accs[19] + w_vec * ot
            return tuple(new)

        return lax.fori_loop(0, n, lbody, accs)

    zeros = tuple(jnp.zeros((16,), jnp.float32) for _ in range(20))

    start(0, 0, LH0, a0, a1, semA)

    @pl.loop(0, ROWS_PER_W)
    def _row(b):
        start(b, LH0, LH1, b0, b1, semB)
        wait(b, 0, LH0, a0, a1, semA)
        accs = accumulate(a0, a1, b, 0, LH0, zeros)

        @pl.when(b < ROWS_PER_W - 1)
        def _():
            start(b + 1, 0, LH0, a0, a1, semA)

        wait(b, LH0, LH1, b0, b1, semB)
        accs = accumulate(b0, b1, b, LH0, LH1, accs)

        for j in range(9):
            repst[b, pl.ds(16 * j, 16)] = accs[j]
        for j in range(9):
            repst[b, pl.ds(HALF + 16 * j, 16)] = accs[9 + j]
        repst[b, pl.ds(136, 16)] = accs[18]
        repst[b, pl.ds(EMB - 16, 16)] = accs[19]

    pltpu.sync_copy(repst, rep_hbm.at[pl.ds(base, ROWS_PER_W)])


BN = 2048


def _tc_body(rep_ref, w_ref, b_ref, out_ref):
    r = jnp.maximum(rep_ref[...], 0.0).astype(jnp.bfloat16)
    w = w_ref[...].astype(jnp.bfloat16)
    acc = lax.dot_general(
        r, w, (((1,), (0,)), ((), ())), preferred_element_type=jnp.float32
    )
    out_ref[...] = acc + b_ref[...]


def _tc_classify(rep, clf_W, clf_b):
    return pl.pallas_call(
        _tc_body,
        grid=(NUM_LABELS // BN,),
        in_specs=[
            pl.BlockSpec((B, EMB), lambda i: (0, 0)),
            pl.BlockSpec((EMB, BN), lambda i: (0, i)),
            pl.BlockSpec((1, BN), lambda i: (0, i)),
        ],
        out_specs=pl.BlockSpec((B, BN), lambda i: (0, i)),
        out_shape=jax.ShapeDtypeStruct((B, NUM_LABELS), jnp.float32),
    )(rep, clf_W.T, clf_b.reshape(1, NUM_LABELS))


def kernel(X, X_w, emb_table, clf_W, clf_b):
    X = X.astype(jnp.int32)
    t0, t1 = _repack(emb_table.T)
    rep = _sc_embed(X, X_w, t0, t1)
    return _tc_classify(rep, clf_W, clf_b)


# speedup vs baseline: 2.2681x; 1.0506x over previous
import dataclasses
import functools

import jax
import jax.numpy as jnp
from jax import lax
from jax.experimental import pallas as pl
from jax.experimental.pallas import tpu as pltpu
from jax.experimental.pallas import tpu_sc as plsc

B = 1024
L = 200
EMB = 300
NUM_LABELS = 32768
VOCAB = 100001

NC = 2
NS = 16
NW = NC * NS
ROWS_PER_W = B // NW
LH0 = 104
LH1 = L - LH0

HALF = 152
S1_OFF = 24

_vector_mesh = plsc.VectorSubcoreMesh(core_axis_name="c", subcore_axis_name="s")

_sc_params = pltpu.CompilerParams()
if "needs_layout_passes" in pltpu.CompilerParams.__dataclass_fields__:
    _sc_params = dataclasses.replace(_sc_params, needs_layout_passes=False)
if "use_tc_tiling_on_sc" in pltpu.CompilerParams.__dataclass_fields__:
    _sc_params = dataclasses.replace(_sc_params, use_tc_tiling_on_sc=False)


RB = 4096


def _repack_body(t_ref, o0_ref, o1_ref):
    v = jnp.transpose(t_ref[...])
    lo = lax.bitcast_convert_type(
        v[:, 0:HALF].astype(jnp.bfloat16), jnp.uint16
    ).astype(jnp.uint32)
    hi = lax.bitcast_convert_type(
        v[:, HALF:2 * HALF].astype(jnp.bfloat16), jnp.uint16
    ).astype(jnp.uint32)
    packed = lax.bitwise_or(lo, lax.shift_left(hi, jnp.uint32(16)))
    o0_ref[...] = packed[:, 0:128]
    o1_ref[...] = packed[:, S1_OFF:S1_OFF + 128]


def _repack(emb_table_t):
    grid = ((VOCAB + RB - 1) // RB,)
    out = jax.ShapeDtypeStruct((VOCAB, 128), jnp.uint32)
    return pl.pallas_call(
        _repack_body,
        grid=grid,
        in_specs=[pl.BlockSpec((2 * HALF, RB), lambda i: (0, i))],
        out_specs=[pl.BlockSpec((RB, 128), lambda i: (i, 0))] * 2,
        out_shape=[out, out],
    )(emb_table_t)


def _splat16(ref, i, j):
    return plsc.load_gather(
        ref, [jnp.full((16,), i, jnp.int32), jnp.full((16,), j, jnp.int32)]
    )


def _unpack32(p):
    b16 = plsc.bitcast(p, jnp.bfloat16)
    return plsc.unpack(
        b16, format=plsc.PackFormat.INTERLEAVED,
        preferred_element_type=jnp.float32,
    )


@functools.partial(
    pl.kernel,
    out_type=jax.ShapeDtypeStruct((B, EMB), jnp.float32),
    mesh=_vector_mesh,
    scratch_types=[
        pltpu.VMEM((ROWS_PER_W, L), jnp.int32),
        pltpu.VMEM((ROWS_PER_W, L), jnp.float32),
        pltpu.VMEM((LH0, 128), jnp.uint32),
        pltpu.VMEM((LH0, 128), jnp.uint32),
        pltpu.VMEM((LH1, 128), jnp.uint32),
        pltpu.VMEM((LH1, 128), jnp.uint32),
        pltpu.VMEM((ROWS_PER_W, EMB), jnp.float32),
        pltpu.SemaphoreType.DMA,
        pltpu.SemaphoreType.DMA,
    ],
    compiler_params=_sc_params,
)
def _sc_embed(x_hbm, w_hbm, t0_hbm, t1_hbm, rep_hbm,
              xv, wv, a0, a1, b0, b1, repst, semA, semB):
    wid = lax.axis_index("s") * NC + lax.axis_index("c")
    base = wid * ROWS_PER_W
    pltpu.sync_copy(x_hbm.at[pl.ds(base, ROWS_PER_W)], xv)
    pltpu.sync_copy(w_hbm.at[pl.ds(base, ROWS_PER_W)], wv)

    def start(b, off, n, g0, g1, sem):
        idx = xv.at[b, pl.ds(off, n)]
        pltpu.make_async_copy(t0_hbm.at[idx], g0, sem).start()
        pltpu.make_async_copy(t1_hbm.at[idx], g1, sem).start()

    def wait(b, off, n, g0, g1, sem):
        idx = xv.at[b, pl.ds(off, n)]
        pltpu.make_async_copy(t0_hbm.at[idx], g0, sem).wait()
        pltpu.make_async_copy(t1_hbm.at[idx], g1, sem).wait()

    def accumulate(g0, g1, b, off, n, accs):
        def lbody(ll, accs):
            l = off + ll
            w_vec = _splat16(wv, b, l)
            new = list(accs)
            for j in range(9):
                if j < 8:
                    p = g0[ll, pl.ds(16 * j, 16)]
                else:
                    p = g1[ll, pl.ds(128 - S1_OFF, 16)]
                e, o = _unpack32(p)
                new[j] = accs[j] + w_vec * e
                new[9 + j] = accs[9 + j] + w_vec * o
            et, _ = _unpack32(g1[ll, pl.ds(136 - S1_OFF, 16)])
            new[18] = accs[18] + w_vec * et
            _, ot = _unpack32(g1[ll, pl.ds(132 - S1_OFF, 16)])
            new[19] = accs[19] + w_vec * ot
            return tuple(new)

        return lax.fori_loop(0, n, lbody, accs)

    zeros = tuple(jnp.zeros((16,), jnp.float32) for _ in range(20))

    start(0, 0, LH0, a0, a1, semA)

    @pl.loop(0, ROWS_PER_W)
    def _row(b):
        start(b, LH0, LH1, b0, b1, semB)
        wait(b, 0, LH0, a0, a1, semA)
        accs = accumulate(a0, a1, b, 0, LH0, zeros)

        @pl.when(b < ROWS_PER_W - 1)
        def _():
            start(b + 1, 0, LH0, a0, a1, semA)

        wait(b, LH0, LH1, b0, b1, semB)
        accs = accumulate(b0, b1, b, LH0, LH1, accs)

        for j in range(9):
            repst[b, pl.ds(16 * j, 16)] = accs[j]
        for j in range(9):
            repst[b, pl.ds(HALF + 16 * j, 16)] = accs[9 + j]
        repst[b, pl.ds(136, 16)] = accs[18]
        repst[b, pl.ds(EMB - 16, 16)] = accs[19]

    pltpu.sync_copy(repst, rep_hbm.at[pl.ds(base, ROWS_PER_W)])


BN = 2048


def _tc_body(rep_ref, w_ref, b_ref, out_ref):
    r = jnp.maximum(rep_ref[...], 0.0).astype(jnp.bfloat16)
    w = w_ref[...].astype(jnp.bfloat16)
    acc = lax.dot_general(
        r, w, (((1,), (0,)), ((), ())), preferred_element_type=jnp.float32
    )
    out_ref[...] = acc + b_ref[...]


def _tc_classify(rep, clf_W, clf_b):
    return pl.pallas_call(
        _tc_body,
        grid=(NUM_LABELS // BN,),
        in_specs=[
            pl.BlockSpec((B, EMB), lambda i: (0, 0)),
            pl.BlockSpec((EMB, BN), lambda i: (0, i)),
            pl.BlockSpec((1, BN), lambda i: (0, i)),
        ],
        out_specs=pl.BlockSpec((B, BN), lambda i: (0, i)),
        out_shape=jax.ShapeDtypeStruct((B, NUM_LABELS), jnp.float32),
    )(rep, clf_W.T, clf_b.reshape(1, NUM_LABELS))


def kernel(X, X_w, emb_table, clf_W, clf_b):
    X = X.astype(jnp.int32)
    t0, t1 = _repack(emb_table.T)
    rep = _sc_embed(X, X_w, t0, t1)
    return _tc_classify(rep, clf_W, clf_b)


# speedup vs baseline: 2.3001x; 1.0141x over previous
import dataclasses
import functools

import jax
import jax.numpy as jnp
from jax import lax
from jax.experimental import pallas as pl
from jax.experimental.pallas import tpu as pltpu
from jax.experimental.pallas import tpu_sc as plsc

B = 1024
L = 200
EMB = 300
NUM_LABELS = 32768
VOCAB = 100001

NC = 2
NS = 16
NW = NC * NS
ROWS_PER_W = B // NW
LH0 = 104
LH1 = L - LH0

HALF = 152
S1_OFF = 24

_vector_mesh = plsc.VectorSubcoreMesh(core_axis_name="c", subcore_axis_name="s")

_sc_params = pltpu.CompilerParams()
if "needs_layout_passes" in pltpu.CompilerParams.__dataclass_fields__:
    _sc_params = dataclasses.replace(_sc_params, needs_layout_passes=False)
if "use_tc_tiling_on_sc" in pltpu.CompilerParams.__dataclass_fields__:
    _sc_params = dataclasses.replace(_sc_params, use_tc_tiling_on_sc=False)


RB = 8192


def _repack_body(t_ref, o0_ref, o1_ref):
    v = jnp.transpose(t_ref[...])
    lo = lax.bitcast_convert_type(
        v[:, 0:HALF].astype(jnp.bfloat16), jnp.uint16
    ).astype(jnp.uint32)
    hi = lax.bitcast_convert_type(
        v[:, HALF:2 * HALF].astype(jnp.bfloat16), jnp.uint16
    ).astype(jnp.uint32)
    packed = lax.bitwise_or(lo, lax.shift_left(hi, jnp.uint32(16)))
    o0_ref[...] = packed[:, 0:128]
    o1_ref[...] = packed[:, S1_OFF:S1_OFF + 128]


def _repack(emb_table_t):
    grid = ((VOCAB + RB - 1) // RB,)
    out = jax.ShapeDtypeStruct((VOCAB, 128), jnp.uint32)
    return pl.pallas_call(
        _repack_body,
        grid=grid,
        in_specs=[pl.BlockSpec((2 * HALF, RB), lambda i: (0, i))],
        out_specs=[pl.BlockSpec((RB, 128), lambda i: (i, 0))] * 2,
        out_shape=[out, out],
    )(emb_table_t)


def _splat16(ref, i, j):
    return plsc.load_gather(
        ref, [jnp.full((16,), i, jnp.int32), jnp.full((16,), j, jnp.int32)]
    )


def _unpack32(p):
    b16 = plsc.bitcast(p, jnp.bfloat16)
    return plsc.unpack(
        b16, format=plsc.PackFormat.INTERLEAVED,
        preferred_element_type=jnp.float32,
    )


@functools.partial(
    pl.kernel,
    out_type=jax.ShapeDtypeStruct((B, EMB), jnp.float32),
    mesh=_vector_mesh,
    scratch_types=[
        pltpu.VMEM((ROWS_PER_W, L), jnp.int32),
        pltpu.VMEM((ROWS_PER_W, L), jnp.float32),
        pltpu.VMEM((LH0, 128), jnp.uint32),
        pltpu.VMEM((LH0, 128), jnp.uint32),
        pltpu.VMEM((LH1, 128), jnp.uint32),
        pltpu.VMEM((LH1, 128), jnp.uint32),
        pltpu.VMEM((ROWS_PER_W, EMB), jnp.float32),
        pltpu.SemaphoreType.DMA,
        pltpu.SemaphoreType.DMA,
    ],
    compiler_params=_sc_params,
)
def _sc_embed(x_hbm, w_hbm, t0_hbm, t1_hbm, rep_hbm,
              xv, wv, a0, a1, b0, b1, repst, semA, semB):
    wid = lax.axis_index("s") * NC + lax.axis_index("c")
    base = wid * ROWS_PER_W
    pltpu.sync_copy(x_hbm.at[pl.ds(base, ROWS_PER_W)], xv)
    pltpu.sync_copy(w_hbm.at[pl.ds(base, ROWS_PER_W)], wv)

    def start(b, off, n, g0, g1, sem):
        idx = xv.at[b, pl.ds(off, n)]
        pltpu.make_async_copy(t0_hbm.at[idx], g0, sem).start()
        pltpu.make_async_copy(t1_hbm.at[idx], g1, sem).start()

    def wait(b, off, n, g0, g1, sem):
        idx = xv.at[b, pl.ds(off, n)]
        pltpu.make_async_copy(t0_hbm.at[idx], g0, sem).wait()
        pltpu.make_async_copy(t1_hbm.at[idx], g1, sem).wait()

    def accumulate(g0, g1, b, off, n, accs):
        def lbody(ll, accs):
            l = off + ll
            w_vec = _splat16(wv, b, l)
            new = list(accs)
            for j in range(9):
                if j < 8:
                    p = g0[ll, pl.ds(16 * j, 16)]
                else:
                    p = g1[ll, pl.ds(128 - S1_OFF, 16)]
                e, o = _unpack32(p)
                new[j] = accs[j] + w_vec * e
                new[9 + j] = accs[9 + j] + w_vec * o
            et, _ = _unpack32(g1[ll, pl.ds(136 - S1_OFF, 16)])
            new[18] = accs[18] + w_vec * et
            _, ot = _unpack32(g1[ll, pl.ds(132 - S1_OFF, 16)])
            new[19] = accs[19] + w_vec * ot
            return tuple(new)

        return lax.fori_loop(0, n, lbody, accs)

    zeros = tuple(jnp.zeros((16,), jnp.float32) for _ in range(20))

    start(0, 0, LH0, a0, a1, semA)

    @pl.loop(0, ROWS_PER_W)
    def _row(b):
        start(b, LH0, LH1, b0, b1, semB)
        wait(b, 0, LH0, a0, a1, semA)
        accs = accumulate(a0, a1, b, 0, LH0, zeros)

        @pl.when(b < ROWS_PER_W - 1)
        def _():
            start(b + 1, 0, LH0, a0, a1, semA)

        wait(b, LH0, LH1, b0, b1, semB)
        accs = accumulate(b0, b1, b, LH0, LH1, accs)

        for j in range(9):
            repst[b, pl.ds(16 * j, 16)] = accs[j]
        for j in range(9):
            repst[b, pl.ds(HALF + 16 * j, 16)] = accs[9 + j]
        repst[b, pl.ds(136, 16)] = accs[18]
        repst[b, pl.ds(EMB - 16, 16)] = accs[19]

    pltpu.sync_copy(repst, rep_hbm.at[pl.ds(base, ROWS_PER_W)])


BN = 4096


def _tc_body(rep_ref, w_ref, b_ref, out_ref):
    r = jnp.maximum(rep_ref[...], 0.0).astype(jnp.bfloat16)
    w = w_ref[...].astype(jnp.bfloat16)
    acc = lax.dot_general(
        r, w, (((1,), (0,)), ((), ())), preferred_element_type=jnp.float32
    )
    out_ref[...] = acc + b_ref[...]


def _tc_classify(rep, clf_W, clf_b):
    return pl.pallas_call(
        _tc_body,
        grid=(NUM_LABELS // BN,),
        in_specs=[
            pl.BlockSpec((B, EMB), lambda i: (0, 0)),
            pl.BlockSpec((EMB, BN), lambda i: (0, i)),
            pl.BlockSpec((1, BN), lambda i: (0, i)),
        ],
        out_specs=pl.BlockSpec((B, BN), lambda i: (0, i)),
        out_shape=jax.ShapeDtypeStruct((B, NUM_LABELS), jnp.float32),
    )(rep, clf_W.T, clf_b.reshape(1, NUM_LABELS))


def kernel(X, X_w, emb_table, clf_W, clf_b):
    X = X.astype(jnp.int32)
    t0, t1 = _repack(emb_table.T)
    rep = _sc_embed(X, X_w, t0, t1)
    return _tc_classify(rep, clf_W, clf_b)
